# baseline (speedup 1.0000x reference)
import jax
import jax.numpy as jnp
from jax import lax
from jax.experimental import pallas as pl
from jax.experimental.pallas import tpu as pltpu
from jax.experimental.pallas import tpu_sc as plsc

_P = 7
_HALF = 16.0
_B, _C, _H, _W = 4, 96, 224, 224
_N = 512
_K = _N * _P * _P
_NC, _NS = 2, 16
_PPB = _K // 8
_QB = 4
_PB = _PPB // _QB
_CH = 1024
_NFULL = 24
_TAILC = _K - _NFULL * _CH
_CPW = _C // 8


def _sc_body(fm_hbm, xmin_h, xspan_h, ymin_h, yspan_h, out_hbm,
             xm_v, xs_v, ym_v, ys_v, tmp0, tmp1, tmp2, tmp3,
             st00, st01, st02, st03, st10, st11, st12, st13,
             plane_v, obuf0, obuf1, sdata, ssem0, ssem1):
    cc = lax.axis_index("c")
    ss = lax.axis_index("s")
    bslot = lax.shift_right_logical(ss, 3)
    tb = ss - bslot * 8
    bb = cc * 2 + bslot
    scale = float(_H - 1)
    ssems = (ssem0, ssem1)
    tmps = (tmp0, tmp1, tmp2, tmp3)
    st = ((st00, st01, st02, st03), (st10, st11, st12, st13))
    obufs = (obuf0, obuf1)

    asl = pl.ds(bb * _N, _N)
    pltpu.sync_copy(xmin_h.at[asl], xm_v)
    pltpu.sync_copy(xspan_h.at[asl], xs_v)
    pltpu.sync_copy(ymin_h.at[asl], ym_v)
    pltpu.sync_copy(yspan_h.at[asl], ys_v)

    for q in range(_QB):
        qbase = tb * _PPB + q * _PB

        def vec_body(vv, carry, qbase=qbase):
            k = qbase + 16 * vv + lax.iota(jnp.int32, 16)
            g = lax.div(k, jnp.full((16,), _P * _P, jnp.int32))
            r = k - g * (_P * _P)
            i = lax.div(r, jnp.full((16,), _P, jnp.int32))
            j = r - i * _P
            xm = plsc.load_gather(xm_v, [g])
            xs = plsc.load_gather(xs_v, [g])
            ym = plsc.load_gather(ym_v, [g])
            ys = plsc.load_gather(ys_v, [g])
            ti = i.astype(jnp.float32) * (1.0 / (_P - 1))
            tj = j.astype(jnp.float32) * (1.0 / (_P - 1))
            px = jnp.clip((xm + xs * ti) * scale, 0.0, scale)
            py = jnp.clip((ym + ys * tj) * scale, 0.0, scale)
            xb = jnp.minimum(px.astype(jnp.int32), _W - 2)
            yb = jnp.minimum(py.astype(jnp.int32), _H - 2)
            sl = pl.ds(16 * vv, 16)
            tmp0[sl] = xb.astype(jnp.float32)
            tmp1[sl] = yb.astype(jnp.float32)
            tmp2[sl] = px - xb.astype(jnp.float32)
            tmp3[sl] = py - yb.astype(jnp.float32)
            return carry

        lax.fori_loop(0, _PB // 16, vec_body, 0)
        for a in range(4):
            pltpu.sync_copy(tmps[a],
                            sdata.at[pl.ds((bslot * 4 + a) * _K + qbase,
                                           _PB)])

    plsc.subcore_barrier()

    def stage(q, slot, n=_CH):
        for a in range(4):
            pltpu.async_copy(
                sdata.at[pl.ds((bslot * 4 + a) * _K + q * _CH, n)],
                st[slot][a].at[pl.ds(0, n)], ssems[slot])

    def drain(q, slot, n=_CH):
        for a in range(4):
            pltpu.make_async_copy(
                sdata.at[pl.ds((bslot * 4 + a) * _K + q * _CH, n)],
                st[slot][a].at[pl.ds(0, n)], ssems[slot]).wait()

    def plane_body(cl, carry):
        ch = tb * _CPW + cl
        pltpu.sync_copy(fm_hbm.at[bb, ch], plane_v)
        prow = bb * _C + ch

        def combine(q, slot, n=_CH):
            drain(q, slot, n)

            def vec_body(vv, carry2):
                sl = pl.ds(16 * vv, 16)
                xbv = st[slot][0][sl].astype(jnp.int32)
                ybv = st[slot][1][sl].astype(jnp.int32)
                dxv = st[slot][2][sl]
                dyv = st[slot][3][sl]
                vlt = plsc.load_gather(plane_v, [ybv, xbv])
                vrt = plsc.load_gather(plane_v, [ybv, xbv + 1])
                vlb = plsc.load_gather(plane_v, [ybv + 1, xbv])
                vrb = plsc.load_gather(plane_v, [ybv + 1, xbv + 1])
                vt = vlt + (vrt - vlt) * dxv
                vb = vlb + (vrb - vlb) * dxv
                obufs[slot][sl] = vt + (vb - vt) * dyv
                return carry2

            lax.fori_loop(0, n // 16, vec_body, 0)
            pltpu.sync_copy(obufs[slot].at[pl.ds(0, n)],
                            out_hbm.at[pl.ds(prow * _K + q * _CH, n)])

        stage(0, 0)
        for p in range(11):
            stage(2 * p + 1, 1)
            combine(2 * p, 0)
            stage(2 * p + 2, 0)
            combine(2 * p + 1, 1)
        stage(23, 1)
        combine(22, 0)
        stage(24, 0, _TAILC)
        combine(23, 1)
        combine(24, 0, _TAILC)
        return carry

    lax.fori_loop(0, _CPW, plane_body, 0)


_sc_call = pl.kernel(
    _sc_body,
    out_type=jax.ShapeDtypeStruct((_B * _C * _K,), jnp.float32),
    mesh=plsc.VectorSubcoreMesh(core_axis_name="c", subcore_axis_name="s"),
    compiler_params=pltpu.CompilerParams(use_tc_tiling_on_sc=True,
                                         needs_layout_passes=False),
    scratch_types=[
        pltpu.VMEM((_N,), jnp.float32),
        pltpu.VMEM((_N,), jnp.float32),
        pltpu.VMEM((_N,), jnp.float32),
        pltpu.VMEM((_N,), jnp.float32),
        pltpu.VMEM((_PB,), jnp.float32),
        pltpu.VMEM((_PB,), jnp.float32),
        pltpu.VMEM((_PB,), jnp.float32),
        pltpu.VMEM((_PB,), jnp.float32),
        pltpu.VMEM((_CH,), jnp.float32),
        pltpu.VMEM((_CH,), jnp.float32),
        pltpu.VMEM((_CH,), jnp.float32),
        pltpu.VMEM((_CH,), jnp.float32),
        pltpu.VMEM((_CH,), jnp.float32),
        pltpu.VMEM((_CH,), jnp.float32),
        pltpu.VMEM((_CH,), jnp.float32),
        pltpu.VMEM((_CH,), jnp.float32),
        pltpu.VMEM((_H, _W), jnp.float32),
        pltpu.VMEM((_CH,), jnp.float32),
        pltpu.VMEM((_CH,), jnp.float32),
        pltpu.VMEM_SHARED((2 * 4 * _K,), jnp.float32),
        pltpu.SemaphoreType.DMA,
        pltpu.SemaphoreType.DMA,
    ],
)


def kernel(feature_map, anchor):
    hl = _HALF / _H
    bmin = jnp.clip(anchor - hl, 0.0, 1.0)
    bmax = jnp.clip(anchor + hl, 0.0, 1.0)
    span = bmax - bmin
    xmin = bmin[:, :, 0].reshape(_B * _N)
    ymin = bmin[:, :, 1].reshape(_B * _N)
    xspan = span[:, :, 0].reshape(_B * _N)
    yspan = span[:, :, 1].reshape(_B * _N)
    out = _sc_call(feature_map, xmin, xspan, ymin, yspan)
    return jnp.transpose(out.reshape(_B, _C, _K), (0, 2, 1))


# speedup vs baseline: 1.9617x; 1.9617x over previous
import jax
import jax.numpy as jnp
from jax import lax
from jax.experimental import pallas as pl
from jax.experimental.pallas import tpu as pltpu
from jax.experimental.pallas import tpu_sc as plsc

_P = 7
_HALF = 16.0
_B, _C, _H, _W = 4, 96, 224, 224
_N = 512
_K = _N * _P * _P
_TOT = _B * _K
_NC, _NS = 2, 16
_NW = _NC * _NS
_PTS_W = _TOT // _NW
_CHUNK = 64
_NCH = _PTS_W // _CHUNK
_APW = _B * _N // _NW
_HW = _H * _W
_CP = 128


_RB = 8


def _tc_relayout_body(fm_ref, out_ref):
    for hh in range(_RB):
        out_ref[pl.ds(_W * hh, _W), :_C] = fm_ref[0, :, hh, :].T


def _relayout(feature_map):
    return pl.pallas_call(
        _tc_relayout_body,
        grid=(_B, _H // _RB),
        in_specs=[pl.BlockSpec((1, _C, _RB, _W), lambda b, h: (b, 0, h, 0))],
        out_specs=pl.BlockSpec((_RB * _W, _CP),
                               lambda b, h: (b * (_H // _RB) + h, 0)),
        out_shape=jax.ShapeDtypeStruct((_B * _HW, _CP), jnp.float32),
    )(feature_map)


def _sc_body(fm_hbm, xmin_h, xspan_h, ymin_h, yspan_h, out_hbm,
             xm_v, xs_v, ym_v, ys_v, idx_v, dx_v, dy_v, gbuf,
             obuf, gsem0, gsem1):
    wid = lax.axis_index("s") * _NC + lax.axis_index("c")
    asl = pl.ds(wid * _APW, _APW)
    pltpu.sync_copy(xmin_h.at[asl], xm_v)
    pltpu.sync_copy(xspan_h.at[asl], xs_v)
    pltpu.sync_copy(ymin_h.at[asl], ym_v)
    pltpu.sync_copy(yspan_h.at[asl], ys_v)
    scale = float(_H - 1)
    gsems = (gsem0, gsem1)

    def compute_and_fire(cc, slot):
        kbase = wid * _PTS_W + cc * _CHUNK
        for v in range(_CHUNK // 16):
            k = kbase + 16 * v + lax.iota(jnp.int32, 16)
            g = lax.div(k, jnp.full((16,), _P * _P, jnp.int32))
            r = k - g * (_P * _P)
            i = lax.div(r, jnp.full((16,), _P, jnp.int32))
            j = r - i * _P
            b = lax.div(g, jnp.full((16,), _N, jnp.int32))
            nloc = g - wid * _APW
            xm = plsc.load_gather(xm_v, [nloc])
            xs = plsc.load_gather(xs_v, [nloc])
            ym = plsc.load_gather(ym_v, [nloc])
            ys = plsc.load_gather(ys_v, [nloc])
            ti = i.astype(jnp.float32) * (1.0 / (_P - 1))
            tj = j.astype(jnp.float32) * (1.0 / (_P - 1))
            px = jnp.clip((xm + xs * ti) * scale, 0.0, scale)
            py = jnp.clip((ym + ys * tj) * scale, 0.0, scale)
            xb = jnp.minimum(px.astype(jnp.int32), _W - 2)
            yb = jnp.minimum(py.astype(jnp.int32), _H - 2)
            dx = px - xb.astype(jnp.float32)
            dy = py - yb.astype(jnp.float32)
            base = b * _HW + yb * _W + xb
            sl = pl.ds(16 * v, 16)
            idx_v[slot, 0, sl] = base
            idx_v[slot, 1, sl] = base + 1
            idx_v[slot, 2, sl] = base + _W
            idx_v[slot, 3, sl] = base + _W + 1
            dx_v[slot, sl] = dx
            dy_v[slot, sl] = dy
        for c in range(4):
            pltpu.async_copy(fm_hbm.at[idx_v.at[slot, c]],
                             gbuf.at[slot, c], gsems[slot])

    def drain(slot):
        for c in range(4):
            pltpu.make_async_copy(fm_hbm.at[idx_v.at[slot, c]],
                                  gbuf.at[slot, c], gsems[slot]).wait()

    def combine_and_store(cc, slot):
        drain(slot)
        kbase = wid * _PTS_W + cc * _CHUNK

        def pt_body(p, pc):
            dxp = dx_v[slot, pl.ds(p, 16)][0]
            dyp = dy_v[slot, pl.ds(p, 16)][0]
            for s in range(_C // 16):
                csl = pl.ds(16 * s, 16)
                vlt = gbuf[slot, 0, p, csl]
                vrt = gbuf[slot, 1, p, csl]
                vlb = gbuf[slot, 2, p, csl]
                vrb = gbuf[slot, 3, p, csl]
                vt = vlt + (vrt - vlt) * dxp
                vb = vlb + (vrb - vlb) * dxp
                obuf[slot, p, csl] = vt + (vb - vt) * dyp
            return pc

        lax.fori_loop(0, _CHUNK, pt_body, 0)
        pltpu.sync_copy(obuf.at[slot], out_hbm.at[pl.ds(kbase, _CHUNK)])

    compute_and_fire(0, 0)

    def pair_body(it, carry):
        cc = 2 * it
        compute_and_fire(cc + 1, 1)
        combine_and_store(cc, 0)
        compute_and_fire(cc + 2, 0)
        combine_and_store(cc + 1, 1)
        return carry

    lax.fori_loop(0, _NCH // 2, pair_body, 0)
    combine_and_store(_NCH - 1, 0)


def kernel(feature_map, anchor):
    fm_rows = _relayout(feature_map)
    hl = _HALF / _H
    bmin = jnp.clip(anchor - hl, 0.0, 1.0)
    bmax = jnp.clip(anchor + hl, 0.0, 1.0)
    span = bmax - bmin
    xmin = bmin[:, :, 0].reshape(_B * _N)
    ymin = bmin[:, :, 1].reshape(_B * _N)
    xspan = span[:, :, 0].reshape(_B * _N)
    yspan = span[:, :, 1].reshape(_B * _N)
    call = pl.kernel(
        _sc_body,
        out_type=jax.ShapeDtypeStruct((_TOT, _C), jnp.float32),
        mesh=plsc.VectorSubcoreMesh(core_axis_name="c", subcore_axis_name="s"),
        compiler_params=pltpu.CompilerParams(use_tc_tiling_on_sc=True,
                                             needs_layout_passes=False),
        scratch_types=[
            pltpu.VMEM((_APW,), jnp.float32),
            pltpu.VMEM((_APW,), jnp.float32),
            pltpu.VMEM((_APW,), jnp.float32),
            pltpu.VMEM((_APW,), jnp.float32),
            pltpu.VMEM((2, 4, _CHUNK), jnp.int32),
            pltpu.VMEM((2, _CHUNK + 16), jnp.float32),
            pltpu.VMEM((2, _CHUNK + 16), jnp.float32),
            pltpu.VMEM((2, 4, _CHUNK, _CP), jnp.float32),
            pltpu.VMEM((2, _CHUNK, _C), jnp.float32),
            pltpu.SemaphoreType.DMA,
            pltpu.SemaphoreType.DMA,
        ],
    )
    out = call(fm_rows, xmin, xspan, ymin, yspan)
    return out.reshape(_B, _K, _C)
